# baseline (speedup 1.0000x reference)
import functools

import jax
import jax.numpy as jnp
from jax import lax
from jax.experimental import pallas as pl
from jax.experimental.pallas import tpu as pltpu
from jax.experimental.pallas import tpu_sc as plsc

NC, NS, LANES = 2, 16, 16
NW = NC * NS

D = 128


def _mesh():
    return plsc.VectorSubcoreMesh(
        core_axis_name="c", subcore_axis_name="s", num_cores=NC, num_subcores=NS
    )


def _wid():
    return lax.axis_index("s") * NC + lax.axis_index("c")


def _sc_gather0_dist2(pa, qb, pos_x, pos_y, pos_z, srcf, dstf, E):
    N = pos_x.shape[0]
    DT = pa.shape[1]
    EPW = E // NW
    CH = 40
    NCHUNK = EPW // CH
    GRP = 2
    NPOST = EPW // LANES - NCHUNK * GRP

    @functools.partial(
        pl.kernel,
        mesh=_mesh(),
        out_type=(
            jax.ShapeDtypeStruct((E, DT), jnp.int32),
            jax.ShapeDtypeStruct((E, DT), jnp.int32),
            jax.ShapeDtypeStruct((E,), jnp.float32),
        ),
        scratch_types=[
            pltpu.VMEM((EPW,), jnp.int32),
            pltpu.VMEM((EPW,), jnp.int32),
            pltpu.VMEM((N,), jnp.float32),
            pltpu.VMEM((N,), jnp.float32),
            pltpu.VMEM((N,), jnp.float32),
            pltpu.VMEM((2, CH, DT), jnp.int32),
            pltpu.VMEM((2, CH, DT), jnp.int32),
            pltpu.VMEM((EPW,), jnp.float32),
        ] + [pltpu.SemaphoreType.DMA] * 4,
        compiler_params=pltpu.CompilerParams(needs_layout_passes=False),
    )
    def k(paH, qbH, pxH, pyH, pzH, srcH, dstH, outA, outB, d2H,
          si, di, px, py, pz, ra, rb, ov, sg0, sg1, ss0, ss1):
        sg = [sg0, sg1]
        ss = [ss0, ss1]
        wid = _wid()
        base = wid * EPW
        pltpu.sync_copy(srcH.at[wid], si)
        pltpu.sync_copy(dstH.at[wid], di)
        pltpu.sync_copy(pxH, px)
        pltpu.sync_copy(pyH, py)
        pltpu.sync_copy(pzH, pz)

        def gissue(i, s):
            pltpu.async_copy(paH.at[si.at[pl.ds(i * CH, CH)]], ra.at[s], sg[s])
            pltpu.async_copy(qbH.at[di.at[pl.ds(i * CH, CH)]], rb.at[s], sg[s])

        def gwait(s):
            pltpu.make_async_copy(paH.at[si.at[pl.ds(0, CH)]], ra.at[s], sg[s]).wait()
            pltpu.make_async_copy(qbH.at[di.at[pl.ds(0, CH)]], rb.at[s], sg[s]).wait()

        def sissue(i, s):
            o = base + i * CH
            pltpu.async_copy(ra.at[s], outA.at[pl.ds(o, CH)], ss[s])
            pltpu.async_copy(rb.at[s], outB.at[pl.ds(o, CH)], ss[s])

        def swait(s):
            pltpu.make_async_copy(ra.at[s], outA.at[pl.ds(0, CH)], ss[s]).wait()
            pltpu.make_async_copy(rb.at[s], outB.at[pl.ds(0, CH)], ss[s]).wait()

        def dist_group(g):
            off = g * LANES
            sv = si[pl.ds(off, LANES)]
            dv = di[pl.ds(off, LANES)]
            dx = plsc.load_gather(px, [sv]) - plsc.load_gather(px, [dv])
            dy = plsc.load_gather(py, [sv]) - plsc.load_gather(py, [dv])
            dz = plsc.load_gather(pz, [sv]) - plsc.load_gather(pz, [dv])
            ov[pl.ds(off, LANES)] = dx * dx + dy * dy + dz * dz + 1e-8

        gissue(0, 0)

        def onechunk(j, s, o):
            @pl.when(j >= 1)
            def _():
                swait(o)

            @pl.when(j + 1 < NCHUNK)
            def _():
                gissue(j + 1, o)

            for kk in range(GRP):
                dist_group(j * GRP + kk)
            gwait(s)
            sissue(j, s)

        def body(j, _):
            @pl.when(j % 2 == 0)
            def _():
                onechunk(j, 0, 1)

            @pl.when(j % 2 == 1)
            def _():
                onechunk(j, 1, 0)

            return 0

        lax.fori_loop(0, NCHUNK, body, 0)
        swait((NCHUNK - 1) % 2)

        def post(g, _):
            dist_group(NCHUNK * GRP + g)
            return 0

        lax.fori_loop(0, NPOST, post, 0)
        pltpu.sync_copy(ov, d2H.at[pl.ds(base, EPW)])

    return k(pa, qb, pos_x, pos_y, pos_z, srcf, dstf)


def _sc_gather2(tableA, tableB, src3, dst3, E):
    DT = tableA.shape[1]
    DTYPE = tableA.dtype
    EPW = E // NW
    CH = src3.shape[2]
    NCHUNK = EPW // CH
    NPAIR = NCHUNK // 2
    SLOTS = 4

    def _slot(i):
        return 2 * ((i // 2) % 2) + (i % 2)

    issued = [sum(1 for i in range(NCHUNK) if _slot(i) == s) for s in range(4)]
    waited = [0] * 4
    for j in range(1, NPAIR):
        for s in ((0, 1) if j % 2 == 1 else (2, 3)):
            waited[s] += 1

    @functools.partial(
        pl.kernel,
        mesh=_mesh(),
        out_type=(
            jax.ShapeDtypeStruct((E, DT), DTYPE),
            jax.ShapeDtypeStruct((E, DT), DTYPE),
        ),
        scratch_types=[
            pltpu.VMEM((NCHUNK, CH), jnp.int32),
            pltpu.VMEM((NCHUNK, CH), jnp.int32),
            pltpu.VMEM((SLOTS, CH, DT), DTYPE),
            pltpu.VMEM((SLOTS, CH, DT), DTYPE),
        ] + [pltpu.SemaphoreType.DMA] * 8,
    )
    def k(tA, tB, src_hbm, dst_hbm, outA, outB, ia, ib, ra, rb,
          sg0, sg1, sg2, sg3, ss0, ss1, ss2, ss3):
        sg = [sg0, sg1, sg2, sg3]
        ss = [ss0, ss1, ss2, ss3]
        wid = _wid()
        base = wid * EPW
        pltpu.sync_copy(src_hbm.at[wid], ia)
        pltpu.sync_copy(dst_hbm.at[wid], ib)

        def gissue(i, s):
            pltpu.async_copy(tA.at[ia.at[i]], ra.at[s], sg[s])
            pltpu.async_copy(tB.at[ib.at[i]], rb.at[s], sg[s])

        def gwait(s):
            pltpu.make_async_copy(tA.at[ia.at[0]], ra.at[s], sg[s]).wait()
            pltpu.make_async_copy(tB.at[ib.at[0]], rb.at[s], sg[s]).wait()

        def sissue(i, s):
            o = base + i * CH
            pltpu.async_copy(ra.at[s], outA.at[pl.ds(o, CH)], ss[s])
            pltpu.async_copy(rb.at[s], outB.at[pl.ds(o, CH)], ss[s])

        def swait(s):
            pltpu.make_async_copy(ra.at[s], outA.at[pl.ds(0, CH)], ss[s]).wait()
            pltpu.make_async_copy(rb.at[s], outB.at[pl.ds(0, CH)], ss[s]).wait()

        gissue(0, 0)
        gissue(1, 1)

        def pair(j, s0, s1, o0, o1):
            i0 = 2 * j
            gwait(s0)
            sissue(i0, s0)
            gwait(s1)
            sissue(i0 + 1, s1)

            @pl.when(j > 0)
            def _():
                swait(o0)
                swait(o1)

            @pl.when(i0 + 2 < NCHUNK)
            def _():
                gissue(i0 + 2, o0)

            @pl.when(i0 + 3 < NCHUNK)
            def _():
                gissue(i0 + 3, o1)

        def body(j, _):
            @pl.when(j % 2 == 0)
            def _():
                pair(j, 0, 1, 2, 3)

            @pl.when(j % 2 == 1)
            def _():
                pair(j, 2, 3, 0, 1)

            return 0

        lax.fori_loop(0, NPAIR, body, 0)
        if NCHUNK % 2 == 1:
            s_tail = _slot(NCHUNK - 1)
            gwait(s_tail)
            sissue(NCHUNK - 1, s_tail)
        for s in range(4):
            for _ in range(issued[s] - waited[s]):
                swait(s)

    return k(tableA, tableB, src3, dst3)


def _sc_scatter(he, dst3, N, E):
    EPW = E // NW
    CH = dst3.shape[2]
    NCHUNK = EPW // CH
    FR = 1000
    NF = N // FR
    ZR = 40

    @functools.partial(
        pl.kernel,
        mesh=_mesh(),
        out_type=jax.ShapeDtypeStruct((NC, N, D), jnp.float32),
        scratch_types=[
            pltpu.VMEM((NCHUNK, CH), jnp.int32),
            pltpu.VMEM((2, CH, D), jnp.float32),
            pltpu.VMEM((ZR, D), jnp.float32),
            pltpu.MemorySpace.VMEM_SHARED((N, D), jnp.float32),
            pltpu.SemaphoreType.DMA,
            pltpu.SemaphoreType.DMA,
            pltpu.SemaphoreType.DMA,
            pltpu.SemaphoreType.DMA,
        ],
    )
    def k(he_hbm, dst_hbm, out_hbm, ii, rows, zb, acc, sl0, sl1, sc0, sc1):
        cid = lax.axis_index("c")
        sid = lax.axis_index("s")
        wid = sid * NC + cid
        base = wid * EPW

        def zvec(i, _):
            for j in range(D // LANES):
                zb[i, pl.ds(j * LANES, LANES)] = jnp.zeros((LANES,), jnp.float32)
            return 0

        lax.fori_loop(0, ZR, zvec, 0)

        @pl.when(sid < NF)
        def _zero():
            for z in range(FR // ZR):
                pltpu.sync_copy(zb, acc.at[pl.ds(sid * FR + z * ZR, ZR)])

        plsc.subcore_barrier()

        pltpu.sync_copy(dst_hbm.at[wid], ii)
        sl = [sl0, sl1]
        sc = [sc0, sc1]

        def lissue(i, s):
            pltpu.async_copy(he_hbm.at[pl.ds(base + i * CH, CH)], rows.at[s], sl[s])

        def lwait(s):
            pltpu.make_async_copy(he_hbm.at[pl.ds(0, CH)], rows.at[s], sl[s]).wait()

        def scissue(i, s):
            pltpu.async_copy(rows.at[s], acc.at[ii.at[i]], sc[s], add=True)

        def scwait(s):
            pltpu.make_async_copy(rows.at[s], acc.at[ii.at[0]], sc[s]).wait()

        lissue(0, 0)

        def onechunk(j, s, o):
            @pl.when(j >= 1)
            def _():
                scwait(o)

            @pl.when(j + 1 < NCHUNK)
            def _():
                lissue(j + 1, o)

            lwait(s)
            scissue(j, s)

        def body(j, _):
            @pl.when(j % 2 == 0)
            def _():
                onechunk(j, 0, 1)

            @pl.when(j % 2 == 1)
            def _():
                onechunk(j, 1, 0)

            return 0

        lax.fori_loop(0, NCHUNK, body, 0)
        scwait((NCHUNK - 1) % 2)
        plsc.subcore_barrier()

        @pl.when(sid < NF)
        def _flush():
            pltpu.sync_copy(acc.at[pl.ds(sid * FR, FR)],
                            out_hbm.at[cid, pl.ds(sid * FR, FR)])

    return k(he, dst3)


def _dot(a, b):
    return jnp.dot(a, b, preferred_element_type=jnp.float32)


def _pack_bf16(hi, lo):
    uh = jax.lax.bitcast_convert_type(hi, jnp.uint32)
    uh = (uh + 0x8000) & jnp.uint32(0xFFFF0000)
    ul = jax.lax.bitcast_convert_type(lo, jnp.uint32)
    ul = (ul + 0x8000) >> 16
    return jax.lax.bitcast_convert_type(uh | ul, jnp.int32)


def _unpack_hi(x):
    u = jax.lax.bitcast_convert_type(x, jnp.uint32)
    return jax.lax.bitcast_convert_type(u & jnp.uint32(0xFFFF0000), jnp.float32)


def _unpack_lo(x):
    u = jax.lax.bitcast_convert_type(x, jnp.uint32)
    return jax.lax.bitcast_convert_type(u << 16, jnp.float32)


def _tc_prep(h_node, wn_emb, wet, web, whe0, ws0, wd0):
    N = h_node.shape[0]
    BLK = 1000
    F = h_node.shape[1]

    def body(h_ref, wn_ref, wet_ref, web_ref, whe_ref, ws_ref, wd_ref,
             hn0_ref, pa_ref, qb_ref):
        h = h_ref[...]
        hn0 = _dot(h, wn_ref[...])
        p = _dot(h, wet_ref[...])
        q = _dot(h, web_ref[...])
        hn0_ref[...] = hn0
        a0 = _dot(p, whe_ref[...]) + _dot(hn0, ws_ref[...])
        b0 = _dot(q, whe_ref[...]) + _dot(hn0, wd_ref[...])
        pa_ref[...] = _pack_bf16(p, a0)
        qb_ref[...] = _pack_bf16(q, b0)

    full = lambda s: pl.BlockSpec(s, lambda i: (0, 0))
    return pl.pallas_call(
        body,
        grid=(N // BLK,),
        in_specs=[
            pl.BlockSpec((BLK, F), lambda i: (i, 0)),
            full((F, D)), full((F, D)), full((F, D)),
            full((D, D)), full((D, D)), full((D, D)),
        ],
        out_specs=[pl.BlockSpec((BLK, D), lambda i: (i, 0))] * 3,
        out_shape=[jax.ShapeDtypeStruct((N, D), jnp.float32),
                   jax.ShapeDtypeStruct((N, D), jnp.int32),
                   jax.ShapeDtypeStruct((N, D), jnp.int32)],
    )(h_node, wn_emb, wet, web, whe0, ws0, wd0)


def _tc_edge0(psa, qdb, d2, wdist):
    E = psa.shape[0]
    EB = 3200

    def body(psa_ref, qdb_ref, d2_ref, w_ref, out_ref):
        psa_b = psa_ref[...]
        qdb_b = qdb_ref[...]
        dist = jnp.sqrt(d2_ref[...])
        m = _unpack_lo(psa_b) + _unpack_lo(qdb_b) + dist * w_ref[...]
        out_ref[...] = (_unpack_hi(psa_b) + _unpack_hi(qdb_b)
                        + jnp.maximum(m, 0.0))

    eb = pl.BlockSpec((EB, D), lambda i: (i, 0))
    return pl.pallas_call(
        body,
        grid=(E // EB,),
        in_specs=[eb, eb,
                  pl.BlockSpec((EB, 1), lambda i: (i, 0)),
                  pl.BlockSpec((1, D), lambda i: (0, 0))],
        out_specs=eb,
        out_shape=jax.ShapeDtypeStruct((E, D), jnp.float32),
    )(psa, qdb, d2, wdist)


def _tc_edge_mm(he, asr, bds, d2, whe, wdist):
    E = he.shape[0]
    EB = 3200

    def body(he_ref, a_ref, b_ref, d2_ref, w_ref, wd_ref, out_ref):
        he_b = he_ref[...]
        dist = jnp.sqrt(d2_ref[...])
        f = jnp.float32
        m = (_dot(he_b, w_ref[...]) + a_ref[...].astype(f)
             + b_ref[...].astype(f) + dist * wd_ref[...])
        out_ref[...] = he_b + jnp.maximum(m, 0.0)

    eb = pl.BlockSpec((EB, D), lambda i: (i, 0))
    full = lambda s: pl.BlockSpec(s, lambda i: (0, 0))
    return pl.pallas_call(
        body,
        grid=(E // EB,),
        in_specs=[eb, eb, eb,
                  pl.BlockSpec((EB, 1), lambda i: (i, 0)),
                  full((D, D)), full((1, D))],
        out_specs=eb,
        out_shape=jax.ShapeDtypeStruct((E, D), jnp.float32),
    )(he, asr, bds, d2, whe, wdist)


def _tc_node(hn, aggp, wnh, wna, wsn, wdn):
    N = hn.shape[0]
    BLK = 1000

    def body(hn_ref, a0_ref, a1_ref, wnh_ref, wna_ref, ws_ref, wd_ref,
             hn_out, a_out, b_out):
        hn_b = hn_ref[...]
        agg = a0_ref[0] + a1_ref[0]
        hn_n = hn_b + jnp.maximum(_dot(hn_b, wnh_ref[...]) + _dot(agg, wna_ref[...]), 0.0)
        hn_out[...] = hn_n
        a_out[...] = _dot(hn_n, ws_ref[...])
        b_out[...] = _dot(hn_n, wd_ref[...])

    nb = pl.BlockSpec((BLK, D), lambda i: (i, 0))
    full = lambda s: pl.BlockSpec(s, lambda i: (0, 0))
    return pl.pallas_call(
        body,
        grid=(N // BLK,),
        in_specs=[nb,
                  pl.BlockSpec((1, BLK, D), lambda i: (0, i, 0)),
                  pl.BlockSpec((1, BLK, D), lambda i: (1, i, 0)),
                  full((D, D)), full((D, D)), full((D, D)), full((D, D))],
        out_specs=[nb, nb, nb],
        out_shape=[jax.ShapeDtypeStruct((N, D), jnp.float32)] * 3,
    )(hn, aggp, aggp, wnh, wna, wsn, wdn)


def _tc_decoder(he3, gs, gd, wd1e, bd1, wd2, bd2, wd3, bd3):
    EH = gs.shape[0]
    EB = 3200
    NB = EH // EB
    NT = wd3.shape[1]

    def body(ha_ref, hb_ref, gs_ref, gd_ref, w1_ref, b1_ref, w2_ref, b2_ref,
             w3_ref, b3_ref, out_ref):
        hsum = ha_ref[...] + hb_ref[...]
        f = jnp.float32
        h1 = jnp.maximum(_dot(hsum, w1_ref[...]) + gs_ref[...].astype(f)
                         + gd_ref[...].astype(f) + b1_ref[...], 0.0)
        h2 = jnp.maximum(_dot(h1, w2_ref[...]) + b2_ref[...], 0.0)
        out_ref[...] = _dot(h2, w3_ref[...]) + b3_ref[...]

    eb = pl.BlockSpec((EB, D), lambda i: (i, 0))
    full = lambda s: pl.BlockSpec(s, lambda i: (0, 0))
    return pl.pallas_call(
        body,
        grid=(NB,),
        in_specs=[pl.BlockSpec((EB, D), lambda i: (i, 0)),
                  pl.BlockSpec((EB, D), lambda i: (i + NB, 0)),
                  eb, eb,
                  full((D, D)), full((1, D)),
                  full((D, D)), full((1, D)),
                  full((D, NT)), full((1, NT))],
        out_specs=pl.BlockSpec((EB, NT), lambda i: (i, 0)),
        out_shape=jax.ShapeDtypeStruct((EH, NT), jnp.float32),
    )(he3, he3, gs, gd, wd1e, bd1, wd2, bd2, wd3, bd3)


def kernel(h_node, pos_node, params, batch_node, edge_index, batch_edge, t):
    N = h_node.shape[0]
    E = edge_index.shape[1]
    EH = E // 2
    F = h_node.shape[1]

    src = edge_index[0]
    dst = edge_index[1]
    EPW = E // NW
    src40 = src.reshape(NW, EPW // 40, 40)
    dst40 = dst.reshape(NW, EPW // 40, 40)
    EPWH = EH // NW
    srch3 = src[:EH].reshape(NW, EPWH // 40, 40)
    dsth3 = dst[:EH].reshape(NW, EPWH // 40, 40)

    wet = params['W_edge_emb'][:F]
    web = params['W_edge_emb'][F:]
    we = [params['We%d' % l] for l in range(3)]
    whe = [w[:D] for w in we]
    ws = [w[D:2 * D] for w in we]
    wd = [w[2 * D:3 * D] for w in we]
    wdist = [w[3 * D:3 * D + 1] for w in we]
    wn = [params['Wn%d' % l] for l in range(3)]
    wnh = [w[:D] for w in wn]
    wna = [w[D:2 * D] for w in wn]
    wd1e = params['Wd1'][:D]
    wd1n = params['Wd1'][D:]
    bd1 = params['bd1'].reshape(1, D)
    bd2 = params['bd2'].reshape(1, D)
    bd3 = params['bd3'].reshape(1, -1)

    hn0, pa, qb = _tc_prep(h_node, params['W_node_emb'], wet, web,
                           whe[0], ws[0], wd[0])

    psa, qdb, d2 = _sc_gather0_dist2(
        pa, qb, pos_node[:, 0], pos_node[:, 1], pos_node[:, 2],
        src.reshape(NW, EPW), dst.reshape(NW, EPW), E)
    d2 = d2.reshape(E, 1)
    he = _tc_edge0(psa, qdb, d2, wdist[0])

    hn = hn0
    for l in range(3):
        aggp = _sc_scatter(he, dst40, N, E)
        if l < 2:
            hn, a, b = _tc_node(hn, aggp, wnh[l], wna[l], ws[l + 1], wd[l + 1])
            asr, bds = _sc_gather2(a, b, src40, dst40, E)
            he = _tc_edge_mm(he, asr, bds, d2, whe[l + 1], wdist[l + 1])
        else:
            hn, g2a, g2b = _tc_node(hn, aggp, wnh[l], wna[l], wd1n, wd1n)
            gs, gd = _sc_gather2(g2a, g2b, srch3, dsth3, EH)

    return _tc_decoder(he, gs, gd, wd1e, bd1, params['Wd2'], bd2,
                       params['Wd3'], bd3)


# speedup vs baseline: 1.0486x; 1.0486x over previous
import functools

import jax
import jax.numpy as jnp
from jax import lax
from jax.experimental import pallas as pl
from jax.experimental.pallas import tpu as pltpu
from jax.experimental.pallas import tpu_sc as plsc

NC, NS, LANES = 2, 16, 16
NW = NC * NS

D = 128


def _mesh():
    return plsc.VectorSubcoreMesh(
        core_axis_name="c", subcore_axis_name="s", num_cores=NC, num_subcores=NS
    )


def _wid():
    return lax.axis_index("s") * NC + lax.axis_index("c")


def _sc_dist2(pos_x, pos_y, pos_z, src2, dst2, E):
    N = pos_x.shape[0]
    EPW = E // NW

    @functools.partial(
        pl.kernel,
        mesh=_mesh(),
        out_type=jax.ShapeDtypeStruct((E,), jnp.float32),
        scratch_types=[
            pltpu.VMEM((N,), jnp.float32),
            pltpu.VMEM((N,), jnp.float32),
            pltpu.VMEM((N,), jnp.float32),
            pltpu.VMEM((EPW,), jnp.int32),
            pltpu.VMEM((EPW,), jnp.int32),
            pltpu.VMEM((EPW,), jnp.float32),
        ],
        compiler_params=pltpu.CompilerParams(needs_layout_passes=False),
    )
    def k(px_hbm, py_hbm, pz_hbm, src_hbm, dst_hbm, out_hbm, px, py, pz, si, di, ov):
        wid = _wid()
        pltpu.sync_copy(px_hbm, px)
        pltpu.sync_copy(py_hbm, py)
        pltpu.sync_copy(pz_hbm, pz)
        pltpu.sync_copy(src_hbm.at[wid], si)
        pltpu.sync_copy(dst_hbm.at[wid], di)

        UNROLL = 5

        def body(j, _):
            for k in range(UNROLL):
                i = j * UNROLL + k
                sv = si[pl.ds(i * LANES, LANES)]
                dv = di[pl.ds(i * LANES, LANES)]
                dx = plsc.load_gather(px, [sv]) - plsc.load_gather(px, [dv])
                dy = plsc.load_gather(py, [sv]) - plsc.load_gather(py, [dv])
                dz = plsc.load_gather(pz, [sv]) - plsc.load_gather(pz, [dv])
                ov[pl.ds(i * LANES, LANES)] = dx * dx + dy * dy + dz * dz + 1e-8
            return 0

        lax.fori_loop(0, EPW // (LANES * UNROLL), body, 0)
        pltpu.sync_copy(ov, out_hbm.at[pl.ds(wid * EPW, EPW)])

    return k(pos_x, pos_y, pos_z, src2, dst2)


def _sc_gather2(tableA, tableB, src3, dst3, E):
    DT = tableA.shape[1]
    DTYPE = tableA.dtype
    EPW = E // NW
    CH = src3.shape[2]
    NCHUNK = EPW // CH

    @functools.partial(
        pl.kernel,
        mesh=_mesh(),
        out_type=(
            jax.ShapeDtypeStruct((E, DT), DTYPE),
            jax.ShapeDtypeStruct((E, DT), DTYPE),
        ),
        scratch_types=[
            pltpu.VMEM((NCHUNK, CH), jnp.int32),
            pltpu.VMEM((NCHUNK, CH), jnp.int32),
            pltpu.VMEM((3, CH, DT), DTYPE),
            pltpu.VMEM((3, CH, DT), DTYPE),
        ] + [pltpu.SemaphoreType.DMA] * 6,
    )
    def k(tA, tB, src_hbm, dst_hbm, outA, outB, ia, ib, ra, rb,
          sg0, sg1, sg2, ss0, ss1, ss2):
        sg = [sg0, sg1, sg2]
        ss = [ss0, ss1, ss2]
        wid = _wid()
        base = wid * EPW
        pltpu.sync_copy(src_hbm.at[wid], ia)
        pltpu.sync_copy(dst_hbm.at[wid], ib)

        def gissue(i, s):
            pltpu.async_copy(tA.at[ia.at[i]], ra.at[s], sg[s])
            pltpu.async_copy(tB.at[ib.at[i]], rb.at[s], sg[s])

        def gwait(s):
            pltpu.make_async_copy(tA.at[ia.at[0]], ra.at[s], sg[s]).wait()
            pltpu.make_async_copy(tB.at[ib.at[0]], rb.at[s], sg[s]).wait()

        def sissue(i, s):
            o = base + i * CH
            pltpu.async_copy(ra.at[s], outA.at[pl.ds(o, CH)], ss[s])
            pltpu.async_copy(rb.at[s], outB.at[pl.ds(o, CH)], ss[s])

        def swait(s):
            pltpu.make_async_copy(ra.at[s], outA.at[pl.ds(0, CH)], ss[s]).wait()
            pltpu.make_async_copy(rb.at[s], outB.at[pl.ds(0, CH)], ss[s]).wait()

        gissue(0, 0)
        gissue(1, 1)

        def onechunk(j, s, so2):
            @pl.when(j >= 1)
            def _():
                swait(so2)

            @pl.when(j + 2 < NCHUNK)
            def _():
                gissue(j + 2, so2)

            gwait(s)
            sissue(j, s)

        def body(j, _):
            @pl.when(j % 3 == 0)
            def _():
                onechunk(j, 0, 2)

            @pl.when(j % 3 == 1)
            def _():
                onechunk(j, 1, 0)

            @pl.when(j % 3 == 2)
            def _():
                onechunk(j, 2, 1)

            return 0

        lax.fori_loop(0, NCHUNK, body, 0)
        swait((NCHUNK - 1) % 3)

    return k(tableA, tableB, src3, dst3)


def _sc_scatter(he, dst3, N, E):
    EPW = E // NW
    CH = dst3.shape[2]
    NCHUNK = EPW // CH
    FR = 1000
    NF = N // FR
    ZR = 40

    @functools.partial(
        pl.kernel,
        mesh=_mesh(),
        out_type=jax.ShapeDtypeStruct((NC, N, D), jnp.float32),
        scratch_types=[
            pltpu.VMEM((NCHUNK, CH), jnp.int32),
            pltpu.VMEM((2, CH, D), jnp.float32),
            pltpu.VMEM((ZR, D), jnp.float32),
            pltpu.MemorySpace.VMEM_SHARED((N, D), jnp.float32),
            pltpu.SemaphoreType.DMA,
            pltpu.SemaphoreType.DMA,
            pltpu.SemaphoreType.DMA,
            pltpu.SemaphoreType.DMA,
        ],
    )
    def k(he_hbm, dst_hbm, out_hbm, ii, rows, zb, acc, sl0, sl1, sc0, sc1):
        cid = lax.axis_index("c")
        sid = lax.axis_index("s")
        wid = sid * NC + cid
        base = wid * EPW

        def zvec(i, _):
            for j in range(D // LANES):
                zb[i, pl.ds(j * LANES, LANES)] = jnp.zeros((LANES,), jnp.float32)
            return 0

        lax.fori_loop(0, ZR, zvec, 0)

        @pl.when(sid < NF)
        def _zero():
            for z in range(FR // ZR):
                pltpu.sync_copy(zb, acc.at[pl.ds(sid * FR + z * ZR, ZR)])

        plsc.subcore_barrier()

        pltpu.sync_copy(dst_hbm.at[wid], ii)
        sl = [sl0, sl1]
        sc = [sc0, sc1]

        def lissue(i, s):
            pltpu.async_copy(he_hbm.at[pl.ds(base + i * CH, CH)], rows.at[s], sl[s])

        def lwait(s):
            pltpu.make_async_copy(he_hbm.at[pl.ds(0, CH)], rows.at[s], sl[s]).wait()

        def scissue(i, s):
            pltpu.async_copy(rows.at[s], acc.at[ii.at[i]], sc[s], add=True)

        def scwait(s):
            pltpu.make_async_copy(rows.at[s], acc.at[ii.at[0]], sc[s]).wait()

        lissue(0, 0)

        def onechunk(j, s, o):
            @pl.when(j >= 1)
            def _():
                scwait(o)

            @pl.when(j + 1 < NCHUNK)
            def _():
                lissue(j + 1, o)

            lwait(s)
            scissue(j, s)

        def body(j, _):
            @pl.when(j % 2 == 0)
            def _():
                onechunk(j, 0, 1)

            @pl.when(j % 2 == 1)
            def _():
                onechunk(j, 1, 0)

            return 0

        lax.fori_loop(0, NCHUNK, body, 0)
        scwait((NCHUNK - 1) % 2)
        plsc.subcore_barrier()

        @pl.when(sid < NF)
        def _flush():
            pltpu.sync_copy(acc.at[pl.ds(sid * FR, FR)],
                            out_hbm.at[cid, pl.ds(sid * FR, FR)])

    return k(he, dst3)


def _dot(a, b):
    return jnp.dot(a, b, preferred_element_type=jnp.float32)


def _pack_bf16(hi, lo):
    uh = jax.lax.bitcast_convert_type(hi, jnp.uint32)
    uh = (uh + 0x8000) & jnp.uint32(0xFFFF0000)
    ul = jax.lax.bitcast_convert_type(lo, jnp.uint32)
    ul = (ul + 0x8000) >> 16
    return jax.lax.bitcast_convert_type(uh | ul, jnp.int32)


def _unpack_hi(x):
    u = jax.lax.bitcast_convert_type(x, jnp.uint32)
    return jax.lax.bitcast_convert_type(u & jnp.uint32(0xFFFF0000), jnp.float32)


def _unpack_lo(x):
    u = jax.lax.bitcast_convert_type(x, jnp.uint32)
    return jax.lax.bitcast_convert_type(u << 16, jnp.float32)


def _tc_prep(h_node, wn_emb, wet, web, whe0, ws0, wd0):
    N = h_node.shape[0]
    BLK = 1000
    F = h_node.shape[1]

    def body(h_ref, wn_ref, wet_ref, web_ref, whe_ref, ws_ref, wd_ref,
             hn0_ref, pa_ref, qb_ref):
        h = h_ref[...]
        hn0 = _dot(h, wn_ref[...])
        p = _dot(h, wet_ref[...])
        q = _dot(h, web_ref[...])
        hn0_ref[...] = hn0
        a0 = _dot(p, whe_ref[...]) + _dot(hn0, ws_ref[...])
        b0 = _dot(q, whe_ref[...]) + _dot(hn0, wd_ref[...])
        pa_ref[...] = _pack_bf16(p, a0)
        qb_ref[...] = _pack_bf16(q, b0)

    full = lambda s: pl.BlockSpec(s, lambda i: (0, 0))
    return pl.pallas_call(
        body,
        grid=(N // BLK,),
        in_specs=[
            pl.BlockSpec((BLK, F), lambda i: (i, 0)),
            full((F, D)), full((F, D)), full((F, D)),
            full((D, D)), full((D, D)), full((D, D)),
        ],
        out_specs=[pl.BlockSpec((BLK, D), lambda i: (i, 0))] * 3,
        out_shape=[jax.ShapeDtypeStruct((N, D), jnp.float32),
                   jax.ShapeDtypeStruct((N, D), jnp.int32),
                   jax.ShapeDtypeStruct((N, D), jnp.int32)],
    )(h_node, wn_emb, wet, web, whe0, ws0, wd0)


def _tc_edge0(psa, qdb, d2, wdist):
    E = psa.shape[0]
    EB = 3200

    def body(psa_ref, qdb_ref, d2_ref, w_ref, out_ref):
        psa_b = psa_ref[...]
        qdb_b = qdb_ref[...]
        dist = jnp.sqrt(d2_ref[...])
        m = _unpack_lo(psa_b) + _unpack_lo(qdb_b) + dist * w_ref[...]
        out_ref[...] = (_unpack_hi(psa_b) + _unpack_hi(qdb_b)
                        + jnp.maximum(m, 0.0))

    eb = pl.BlockSpec((EB, D), lambda i: (i, 0))
    return pl.pallas_call(
        body,
        grid=(E // EB,),
        in_specs=[eb, eb,
                  pl.BlockSpec((EB, 1), lambda i: (i, 0)),
                  pl.BlockSpec((1, D), lambda i: (0, 0))],
        out_specs=eb,
        out_shape=jax.ShapeDtypeStruct((E, D), jnp.float32),
    )(psa, qdb, d2, wdist)


def _tc_edge_mm(he, asr, bds, d2, whe, wdist):
    E = he.shape[0]
    EB = 3200

    def body(he_ref, a_ref, b_ref, d2_ref, w_ref, wd_ref, out_ref):
        he_b = he_ref[...]
        dist = jnp.sqrt(d2_ref[...])
        f = jnp.float32
        m = (_dot(he_b, w_ref[...]) + a_ref[...].astype(f)
             + b_ref[...].astype(f) + dist * wd_ref[...])
        out_ref[...] = he_b + jnp.maximum(m, 0.0)

    eb = pl.BlockSpec((EB, D), lambda i: (i, 0))
    full = lambda s: pl.BlockSpec(s, lambda i: (0, 0))
    return pl.pallas_call(
        body,
        grid=(E // EB,),
        in_specs=[eb, eb, eb,
                  pl.BlockSpec((EB, 1), lambda i: (i, 0)),
                  full((D, D)), full((1, D))],
        out_specs=eb,
        out_shape=jax.ShapeDtypeStruct((E, D), jnp.float32),
    )(he, asr, bds, d2, whe, wdist)


def _tc_node(hn, aggp, wnh, wna, wsn, wdn):
    N = hn.shape[0]
    BLK = 1000

    def body(hn_ref, a0_ref, a1_ref, wnh_ref, wna_ref, ws_ref, wd_ref,
             hn_out, a_out, b_out):
        hn_b = hn_ref[...]
        agg = a0_ref[0] + a1_ref[0]
        hn_n = hn_b + jnp.maximum(_dot(hn_b, wnh_ref[...]) + _dot(agg, wna_ref[...]), 0.0)
        hn_out[...] = hn_n
        a_out[...] = _dot(hn_n, ws_ref[...])
        b_out[...] = _dot(hn_n, wd_ref[...])

    nb = pl.BlockSpec((BLK, D), lambda i: (i, 0))
    full = lambda s: pl.BlockSpec(s, lambda i: (0, 0))
    return pl.pallas_call(
        body,
        grid=(N // BLK,),
        in_specs=[nb,
                  pl.BlockSpec((1, BLK, D), lambda i: (0, i, 0)),
                  pl.BlockSpec((1, BLK, D), lambda i: (1, i, 0)),
                  full((D, D)), full((D, D)), full((D, D)), full((D, D))],
        out_specs=[nb, nb, nb],
        out_shape=[jax.ShapeDtypeStruct((N, D), jnp.float32)] * 3,
    )(hn, aggp, aggp, wnh, wna, wsn, wdn)


def _tc_decoder(he3, gs, gd, wd1e, bd1, wd2, bd2, wd3, bd3):
    EH = gs.shape[0]
    EB = 3200
    NB = EH // EB
    NT = wd3.shape[1]

    def body(ha_ref, hb_ref, gs_ref, gd_ref, w1_ref, b1_ref, w2_ref, b2_ref,
             w3_ref, b3_ref, out_ref):
        hsum = ha_ref[...] + hb_ref[...]
        f = jnp.float32
        h1 = jnp.maximum(_dot(hsum, w1_ref[...]) + gs_ref[...].astype(f)
                         + gd_ref[...].astype(f) + b1_ref[...], 0.0)
        h2 = jnp.maximum(_dot(h1, w2_ref[...]) + b2_ref[...], 0.0)
        out_ref[...] = _dot(h2, w3_ref[...]) + b3_ref[...]

    eb = pl.BlockSpec((EB, D), lambda i: (i, 0))
    full = lambda s: pl.BlockSpec(s, lambda i: (0, 0))
    return pl.pallas_call(
        body,
        grid=(NB,),
        in_specs=[pl.BlockSpec((EB, D), lambda i: (i, 0)),
                  pl.BlockSpec((EB, D), lambda i: (i + NB, 0)),
                  eb, eb,
                  full((D, D)), full((1, D)),
                  full((D, D)), full((1, D)),
                  full((D, NT)), full((1, NT))],
        out_specs=pl.BlockSpec((EB, NT), lambda i: (i, 0)),
        out_shape=jax.ShapeDtypeStruct((EH, NT), jnp.float32),
    )(he3, he3, gs, gd, wd1e, bd1, wd2, bd2, wd3, bd3)


def kernel(h_node, pos_node, params, batch_node, edge_index, batch_edge, t):
    N = h_node.shape[0]
    E = edge_index.shape[1]
    EH = E // 2
    F = h_node.shape[1]

    src = edge_index[0]
    dst = edge_index[1]
    EPW = E // NW
    src40 = src.reshape(NW, EPW // 40, 40)
    dst40 = dst.reshape(NW, EPW // 40, 40)
    src80 = src.reshape(NW, EPW // 80, 80)
    dst80 = dst.reshape(NW, EPW // 80, 80)
    EPWH = EH // NW
    srch3 = src[:EH].reshape(NW, EPWH // 40, 40)
    dsth3 = dst[:EH].reshape(NW, EPWH // 40, 40)

    wet = params['W_edge_emb'][:F]
    web = params['W_edge_emb'][F:]
    we = [params['We%d' % l] for l in range(3)]
    whe = [w[:D] for w in we]
    ws = [w[D:2 * D] for w in we]
    wd = [w[2 * D:3 * D] for w in we]
    wdist = [w[3 * D:3 * D + 1] for w in we]
    wn = [params['Wn%d' % l] for l in range(3)]
    wnh = [w[:D] for w in wn]
    wna = [w[D:2 * D] for w in wn]
    wd1e = params['Wd1'][:D]
    wd1n = params['Wd1'][D:]
    bd1 = params['bd1'].reshape(1, D)
    bd2 = params['bd2'].reshape(1, D)
    bd3 = params['bd3'].reshape(1, -1)

    d2 = _sc_dist2(pos_node[:, 0], pos_node[:, 1], pos_node[:, 2],
                   src.reshape(NW, EPW), dst.reshape(NW, EPW), E).reshape(E, 1)

    hn0, pa, qb = _tc_prep(h_node, params['W_node_emb'], wet, web,
                           whe[0], ws[0], wd[0])

    psa, qdb = _sc_gather2(pa, qb, src80, dst80, E)
    he = _tc_edge0(psa, qdb, d2, wdist[0])

    hn = hn0
    for l in range(3):
        aggp = _sc_scatter(he, dst40, N, E)
        if l < 2:
            hn, a, b = _tc_node(hn, aggp, wnh[l], wna[l], ws[l + 1], wd[l + 1])
            asr, bds = _sc_gather2(a, b, src80, dst80, E)
            he = _tc_edge_mm(he, asr, bds, d2, whe[l + 1], wdist[l + 1])
        else:
            hn, g2a, g2b = _tc_node(hn, aggp, wnh[l], wna[l], wd1n, wd1n)
            gs, gd = _sc_gather2(g2a, g2b, srch3, dsth3, EH)

    return _tc_decoder(he, gs, gd, wd1e, bd1, params['Wd2'], bd2,
                       params['Wd3'], bd3)
